# baseline (speedup 1.0000x reference)
import functools

import jax
import jax.numpy as jnp
from jax.experimental import pallas as pl
from jax.experimental.pallas import tpu as pltpu

_WCHUNK = 4


def _read_chunks(P):
    bounds, s = [], 0
    while s < P:
        step = 2 if s < P - 2 else 1
        bounds.append((s, s + step))
        s += step
    return bounds


def _gc_kernel_body(adj_hbm, x_hbm, w_hbm, out_hbm,
                    x_buf, w_buf, adj_buf, o_buf,
                    adj_sem, x_sem, w_sem, out_sem, *, W, P):
    chunks = _read_chunks(P)
    n_wchunks = P // _WCHUNK
    chunk_of_pair = {}
    for c, (s, e) in enumerate(chunks):
        for k in range(s, e):
            chunk_of_pair[k] = c

    def adj_copy(c):
        s, e = chunks[c]
        sl = pl.ds(s, e - s)
        return pltpu.make_async_copy(adj_hbm.at[sl], adj_buf.at[sl],
                                     adj_sem.at[c])

    def out_copy(c):
        sl = pl.ds(c * _WCHUNK, _WCHUNK)
        return pltpu.make_async_copy(o_buf.at[sl], out_hbm.at[sl],
                                     out_sem.at[c])

    x_copy = pltpu.make_async_copy(x_hbm, x_buf, x_sem)
    w_copy = pltpu.make_async_copy(w_hbm, w_buf, w_sem)

    x_copy.start()
    w_copy.start()
    for c in range(len(chunks)):
        adj_copy(c).start()
    x_copy.wait()
    w_copy.wait()

    waited = set()
    for k in range(P):
        c = chunk_of_pair[k]
        if c not in waited:
            adj_copy(c).wait()
            waited.add(c)
        xw = jnp.dot(x_buf[k], w_buf[k % W],
                     preferred_element_type=jnp.float32)
        o_buf[k] = jnp.dot(adj_buf[k], xw,
                           preferred_element_type=jnp.float32)
        if (k + 1) % _WCHUNK == 0:
            out_copy(k // _WCHUNK).start()

    for c in range(n_wchunks):
        out_copy(c).wait()


def kernel(adjacency, nodes, weights):
    B, W, N, _ = adjacency.shape
    Fin = nodes.shape[-1]
    Fout = weights.shape[-1]
    itemsize = jnp.dtype(adjacency.dtype).itemsize
    P = B * W

    flops = 2 * B * W * (N * N * Fout + N * Fin * Fout)
    bytes_accessed = itemsize * (adjacency.size + nodes.size + weights.size
                                 + B * W * N * Fout)
    cost = pl.CostEstimate(flops=flops, transcendentals=0,
                           bytes_accessed=bytes_accessed)

    body = functools.partial(_gc_kernel_body, W=W, P=P)

    out_flat = pl.pallas_call(
        body,
        out_shape=jax.ShapeDtypeStruct((P, N, Fout), nodes.dtype),
        grid=(1,),
        in_specs=[
            pl.BlockSpec(memory_space=pl.ANY),
            pl.BlockSpec(memory_space=pl.ANY),
            pl.BlockSpec(memory_space=pl.ANY),
        ],
        out_specs=pl.BlockSpec(memory_space=pl.ANY),
        scratch_shapes=[
            pltpu.VMEM((P, N, Fin), jnp.float32),
            pltpu.VMEM((W, Fin, Fout), jnp.float32),
            pltpu.VMEM((P, N, N), jnp.float32),
            pltpu.VMEM((P, N, Fout), jnp.float32),
            pltpu.SemaphoreType.DMA((len(_read_chunks(P)),)),
            pltpu.SemaphoreType.DMA,
            pltpu.SemaphoreType.DMA,
            pltpu.SemaphoreType.DMA((P // _WCHUNK,)),
        ],
        compiler_params=pltpu.CompilerParams(
            dimension_semantics=("arbitrary",),
            vmem_limit_bytes=52 * 1024 * 1024,
        ),
        cost_estimate=cost,
    )(adjacency.reshape(P, N, N), nodes.reshape(P, N, Fin), weights)

    return out_flat.reshape(B, W, N, Fout)


# speedup vs baseline: 1.0280x; 1.0280x over previous
import functools

import jax
import jax.numpy as jnp
from jax.experimental import pallas as pl
from jax.experimental.pallas import tpu as pltpu

_WCHUNK = 4


def _read_chunks(P):
    return [(s, s + 2) for s in range(0, P, 2)]


def _gc_kernel_body(adj_hbm, x_hbm, w_hbm, out_hbm,
                    x_buf, w_buf, adj_buf, o_buf,
                    adj_sem, x_sem, w_sem, out_sem, *, W, P):
    chunks = _read_chunks(P)
    n_wchunks = P // _WCHUNK
    chunk_of_pair = {}
    for c, (s, e) in enumerate(chunks):
        for k in range(s, e):
            chunk_of_pair[k] = c

    def adj_copy(c):
        s, e = chunks[c]
        sl = pl.ds(s, e - s)
        return pltpu.make_async_copy(adj_hbm.at[sl], adj_buf.at[sl],
                                     adj_sem.at[c])

    def out_copy(c):
        sl = pl.ds(c * _WCHUNK, _WCHUNK)
        return pltpu.make_async_copy(o_buf.at[sl], out_hbm.at[sl],
                                     out_sem.at[c])

    x_copy = pltpu.make_async_copy(x_hbm, x_buf, x_sem)
    w_copy = pltpu.make_async_copy(w_hbm, w_buf, w_sem)

    x_copy.start()
    w_copy.start()
    for c in range(len(chunks)):
        adj_copy(c).start()
    x_copy.wait()
    w_copy.wait()

    waited = set()
    for k in range(P):
        c = chunk_of_pair[k]
        if c not in waited:
            adj_copy(c).wait()
            waited.add(c)
        xw = jnp.dot(x_buf[k], w_buf[k % W],
                     preferred_element_type=jnp.float32)
        o_buf[k] = jnp.dot(adj_buf[k], xw,
                           preferred_element_type=jnp.float32)
        if (k + 1) % _WCHUNK == 0:
            out_copy(k // _WCHUNK).start()

    for c in range(n_wchunks):
        out_copy(c).wait()


def kernel(adjacency, nodes, weights):
    B, W, N, _ = adjacency.shape
    Fin = nodes.shape[-1]
    Fout = weights.shape[-1]
    itemsize = jnp.dtype(adjacency.dtype).itemsize
    P = B * W

    flops = 2 * B * W * (N * N * Fout + N * Fin * Fout)
    bytes_accessed = itemsize * (adjacency.size + nodes.size + weights.size
                                 + B * W * N * Fout)
    cost = pl.CostEstimate(flops=flops, transcendentals=0,
                           bytes_accessed=bytes_accessed)

    body = functools.partial(_gc_kernel_body, W=W, P=P)

    out_flat = pl.pallas_call(
        body,
        out_shape=jax.ShapeDtypeStruct((P, N, Fout), nodes.dtype),
        grid=(1,),
        in_specs=[
            pl.BlockSpec(memory_space=pl.ANY),
            pl.BlockSpec(memory_space=pl.ANY),
            pl.BlockSpec(memory_space=pl.ANY),
        ],
        out_specs=pl.BlockSpec(memory_space=pl.ANY),
        scratch_shapes=[
            pltpu.VMEM((P, N, Fin), jnp.float32),
            pltpu.VMEM((W, Fin, Fout), jnp.float32),
            pltpu.VMEM((P, N, N), jnp.float32),
            pltpu.VMEM((P, N, Fout), jnp.float32),
            pltpu.SemaphoreType.DMA((len(_read_chunks(P)),)),
            pltpu.SemaphoreType.DMA,
            pltpu.SemaphoreType.DMA,
            pltpu.SemaphoreType.DMA((P // _WCHUNK,)),
        ],
        compiler_params=pltpu.CompilerParams(
            dimension_semantics=("arbitrary",),
            vmem_limit_bytes=52 * 1024 * 1024,
        ),
        cost_estimate=cost,
    )(adjacency.reshape(P, N, N), nodes.reshape(P, N, Fin), weights)

    return out_flat.reshape(B, W, N, Fout)
